# Initial kernel scaffold; baseline (speedup 1.0000x reference)
import jax
import jax.numpy as jnp
from jax.experimental import pallas as pl


def kernel(hidden_states, gate_w, Wg, Wu, Wd):
    raise NotImplementedError("write your pallas kernel here")


# speedup vs baseline: 2.2637x; 2.2637x over previous
import functools

import jax
import jax.numpy as jnp
from jax import lax
from jax.experimental import pallas as pl
from jax.experimental.pallas import tpu as pltpu

H = 768
F = 768
E = 64
T = 256


def _logits_body(x_ref, gw_ref, lt_ref):
    lt_ref[...] = lax.dot_general(
        gw_ref[...], x_ref[...],
        dimension_numbers=(((0,), (1,)), ((), ())),
        preferred_element_type=jnp.float32,
    )


def _routing_body(lt_ref, comb_ref):
    lt = lt_ref[...]
    iota_e = lax.broadcasted_iota(jnp.int32, (E, T), 0)
    m1 = jnp.max(lt, axis=0, keepdims=True)
    i1 = jnp.min(jnp.where(lt == m1, iota_e, E), axis=0, keepdims=True)
    mask1 = iota_e == i1
    lt2 = jnp.where(mask1, -jnp.inf, lt)
    m2 = jnp.max(lt2, axis=0, keepdims=True)
    i2 = jnp.min(jnp.where(lt2 == m2, iota_e, E), axis=0, keepdims=True)
    mask2 = iota_e == i2
    w1 = 1.0 / (1.0 + jnp.exp(m2 - m1))
    combT = jnp.where(mask1, w1, 0.0) + jnp.where(mask2, 1.0 - w1, 0.0)
    comb_ref[...] = combT.T


def _moe_body(xb_ref, comb_ref, wg_ref, wu_ref, wd_ref, out_ref):
    e = pl.program_id(0)

    @pl.when(e == 0)
    def _():
        out_ref[...] = jnp.zeros_like(out_ref)

    xb = xb_ref[...]
    wg = wg_ref[0].astype(jnp.bfloat16)
    wu = wu_ref[0].astype(jnp.bfloat16)
    g = lax.dot(xb, wg, preferred_element_type=jnp.float32)
    u = lax.dot(xb, wu, preferred_element_type=jnp.float32)
    h = (g * lax.logistic(g)) * u
    wd = wd_ref[0].astype(jnp.bfloat16)
    y = lax.dot(h.astype(jnp.bfloat16), wd, preferred_element_type=jnp.float32)
    onehot = (lax.broadcasted_iota(jnp.int32, (E, 1), 0) == e).astype(jnp.float32)
    ce = lax.dot(comb_ref[...], onehot, preferred_element_type=jnp.float32)
    out_ref[...] += ce * y


def kernel(hidden_states, gate_w, Wg, Wu, Wd):
    x = hidden_states.reshape(-1, H)

    logitsT = pl.pallas_call(
        _logits_body,
        out_shape=jax.ShapeDtypeStruct((E, T), jnp.float32),
    )(x, gate_w)

    comb = pl.pallas_call(
        _routing_body,
        out_shape=jax.ShapeDtypeStruct((T, E), jnp.float32),
    )(logitsT)

    xb = x.astype(jnp.bfloat16)
    out = pl.pallas_call(
        _moe_body,
        grid=(E,),
        in_specs=[
            pl.BlockSpec((T, H), lambda e: (0, 0)),
            pl.BlockSpec((T, E), lambda e: (0, 0)),
            pl.BlockSpec((1, H, F), lambda e: (e, 0, 0)),
            pl.BlockSpec((1, H, F), lambda e: (e, 0, 0)),
            pl.BlockSpec((1, F, H), lambda e: (e, 0, 0)),
        ],
        out_specs=pl.BlockSpec((T, H), lambda e: (0, 0)),
        out_shape=jax.ShapeDtypeStruct((T, H), jnp.float32),
        compiler_params=pltpu.CompilerParams(
            dimension_semantics=("arbitrary",),
        ),
    )(xb, comb, Wg, Wu, Wd)
    return out
